# baseline (speedup 1.0000x reference)
import functools

import jax
import jax.numpy as jnp
from jax import lax
from jax.experimental import pallas as pl
from jax.experimental.pallas import tpu as pltpu
from jax.experimental.pallas import tpu_sc as plsc

_NC = 2
_NS = 16
_NW = _NC * _NS
_LANES = 16
_NBUF = 4
_HGS = ((0, 104), (104, 96))


@functools.lru_cache(maxsize=None)
def _make_sc_gather(V, D, B, H):
    b_per_w = B // _NW
    scale = float(D) ** 0.5
    mesh = plsc.VectorSubcoreMesh(core_axis_name="c", subcore_axis_name="s")

    @functools.partial(
        pl.kernel,
        mesh=mesh,
        out_type=jax.ShapeDtypeStruct((B, H, D), jnp.float32),
        scratch_types=[
            pltpu.VMEM((b_per_w, H), jnp.int32),
            pltpu.VMEM((_NBUF, H, D), jnp.float32),
            pltpu.SemaphoreType.DMA((_NBUF,)),
            pltpu.SemaphoreType.DMA((_NBUF,)),
        ],
        compiler_params=pltpu.CompilerParams(use_tc_tiling_on_sc=False),
    )
    def sc_kernel(x_hbm, table_hbm, out_hbm, idx_v, rows_v, gsem, wsem):
        wid = lax.axis_index("s") * _NC + lax.axis_index("c")
        base = wid * b_per_w
        pltpu.sync_copy(x_hbm.at[pl.ds(base, b_per_w)], idx_v)

        def gather_start(bl, s):
            for off, n in _HGS:
                pltpu.async_copy(
                    table_hbm.at[idx_v.at[bl, pl.ds(off, n)]],
                    rows_v.at[s, pl.ds(off, n)], gsem.at[s])

        def gather_wait(bl, s):
            for off, n in _HGS:
                pltpu.make_async_copy(
                    table_hbm.at[idx_v.at[bl, pl.ds(off, n)]],
                    rows_v.at[s, pl.ds(off, n)], gsem.at[s]).wait()

        def write_start(bl, s):
            pltpu.async_copy(rows_v.at[s], out_hbm.at[base + bl], wsem.at[s])

        def write_wait(s):
            pltpu.make_async_copy(rows_v.at[s], out_hbm.at[0],
                                  wsem.at[s]).wait()

        for s in range(_NBUF - 1):
            gather_start(s, s)

        @pl.loop(0, b_per_w, step=_NBUF)
        def _group(b0):
            for j in range(_NBUF):
                bl = b0 + j
                s = j
                s2 = (j + _NBUF - 1) % _NBUF
                bn = bl + _NBUF - 1

                @pl.when(bn < b_per_w)
                def _():
                    @pl.when(bl > 0)
                    def _():
                        write_wait(s2)
                    gather_start(bn, s2)

                gather_wait(bl, s)

                @plsc.parallel_loop(0, H, unroll=4)
                def _row(r):
                    for c in range(D // _LANES):
                        sl = pl.ds(c * _LANES, _LANES)
                        rows_v[s, r, sl] = rows_v[s, r, sl] * scale

                write_start(bl, s)

        for s in range(_NBUF):
            write_wait(s)

    return sc_kernel


def kernel(x, table):
    B, H = x.shape
    V, D = table.shape
    sc = _make_sc_gather(V, D, B, H)
    return sc(x.astype(jnp.int32), table)
